# baseline (speedup 1.0000x reference)
import functools

import jax
import jax.numpy as jnp
from jax import lax
from jax.experimental import pallas as pl
from jax.experimental.pallas import tpu as pltpu
from jax.experimental.pallas import tpu_sc as plsc

BATCH = 1024
IN = 100000
OUT = 1024

NW = 32
GROUP_ROWS = 8
NGROUPS = BATCH // GROUP_ROWS
GPW = NGROUPS // NW
CQ = 4
CHW = CQ * OUT
NCH = 24
SC_COLS = NCH * CHW
TAIL_COLS = IN - SC_COLS

_mesh = plsc.VectorSubcoreMesh(core_axis_name="c", subcore_axis_name="s")


@functools.partial(
    pl.kernel,
    mesh=_mesh,
    out_type=jax.ShapeDtypeStruct((BATCH, OUT), jnp.float32),
    scratch_types=[
        pltpu.VMEM((GROUP_ROWS, CHW), jnp.float32),
        pltpu.VMEM((GROUP_ROWS, CHW), jnp.float32),
        pltpu.VMEM((GROUP_ROWS, OUT), jnp.float32),
        pltpu.SemaphoreType.DMA,
        pltpu.SemaphoreType.DMA,
        pltpu.SemaphoreType.DMA,
    ],
    compiler_params=pltpu.CompilerParams(use_tc_tiling_on_sc=True),
)
def _sc_kernel(x_hbm, out_hbm, buf0, buf1, acc, sem0, sem1, sema):
    bufs = (buf0, buf1)
    sems = (sem0, sem1)
    wid = lax.axis_index("s") * 2 + lax.axis_index("c")

    def start_chunk(g, c, b):
        pltpu.async_copy(
            x_hbm.at[pl.ds(g * GROUP_ROWS, GROUP_ROWS), pl.ds(c * CHW, CHW)],
            bufs[b], sems[b])

    def wait_chunk(b):
        pltpu.make_async_copy(
            x_hbm.at[pl.ds(0, GROUP_ROWS), pl.ds(0, CHW)],
            bufs[b], sems[b]).wait()

    def accum_chunk(b):
        buf = bufs[b]
        for s in range(GROUP_ROWS):
            def h_body(h, carry):
                v = buf[s, pl.ds(h * 16, 16)]
                for p in range(1, CQ):
                    v = v + buf[s, pl.ds(p * OUT + h * 16, 16)]
                plsc.addupdate(acc.at[s, pl.ds(h * 16, 16)], v)
                return carry

            lax.fori_loop(0, OUT // 16, h_body, 0, unroll=8)

    zero16 = jnp.zeros((16,), jnp.float32)

    def group_body(gi, carry):
        g = wid * GPW + gi
        start_chunk(g, 0, 0)
        start_chunk(g, 1, 1)
        for s in range(GROUP_ROWS):
            def z_body(h, carry0):
                acc[s, pl.ds(h * 16, 16)] = zero16
                return carry0

            lax.fori_loop(0, OUT // 16, z_body, 0, unroll=8)

        def chunk_body(i, carry2):
            for b in range(2):
                c = 2 * i + b
                wait_chunk(b)
                accum_chunk(b)

                @pl.when(c + 2 < NCH)
                def _():
                    start_chunk(g, c + 2, b)

            return carry2

        lax.fori_loop(0, NCH // 2, chunk_body, 0, unroll=False)
        pltpu.sync_copy(acc, out_hbm.at[pl.ds(g * GROUP_ROWS, GROUP_ROWS), :])
        return carry

    lax.fori_loop(0, GPW, group_body, 0, unroll=False)


_TAIL_BBLK = 256


def _tail_body(x_ref, o_ref):
    x = x_ref[...]
    col = jax.lax.broadcasted_iota(jnp.int32, x.shape, 1)
    o_ref[...] = x[:, 0:OUT] + jnp.where(col < TAIL_COLS, x, 0.0)[:, OUT:2 * OUT]


def _tail_partial(x):
    return pl.pallas_call(
        _tail_body,
        grid=(BATCH // _TAIL_BBLK,),
        in_specs=[pl.BlockSpec((_TAIL_BBLK, 2 * OUT),
                               lambda i: (i, SC_COLS // (2 * OUT)))],
        out_specs=pl.BlockSpec((_TAIL_BBLK, OUT), lambda i: (i, 0)),
        out_shape=jax.ShapeDtypeStruct((BATCH, OUT), jnp.float32),
        compiler_params=pltpu.CompilerParams(
            dimension_semantics=("parallel",),
        ),
    )(x)


@jax.jit
def kernel(probability_distribution):
    tail = _tail_partial(probability_distribution)
    main = _sc_kernel(probability_distribution)
    return main + tail


# speedup vs baseline: 1.1028x; 1.1028x over previous
import functools

import jax
import jax.numpy as jnp
from jax import lax
from jax.experimental import pallas as pl
from jax.experimental.pallas import tpu as pltpu
from jax.experimental.pallas import tpu_sc as plsc

BATCH = 1024
IN = 100000
OUT = 1024

NW = 32
GROUP_ROWS = 8
SC_ROWS = 512
GPW = SC_ROWS // GROUP_ROWS // NW
CQ = 4
CHW = CQ * OUT
NCH = 24
SC_COLS = NCH * CHW
TAIL_COLS = IN - SC_COLS

_mesh = plsc.VectorSubcoreMesh(core_axis_name="c", subcore_axis_name="s")


@functools.partial(
    pl.kernel,
    mesh=_mesh,
    out_type=jax.ShapeDtypeStruct((SC_ROWS, OUT), jnp.float32),
    scratch_types=[
        pltpu.VMEM((GROUP_ROWS, CHW), jnp.float32),
        pltpu.VMEM((GROUP_ROWS, CHW), jnp.float32),
        pltpu.VMEM((GROUP_ROWS, OUT), jnp.float32),
        pltpu.SemaphoreType.DMA,
        pltpu.SemaphoreType.DMA,
    ],
    compiler_params=pltpu.CompilerParams(use_tc_tiling_on_sc=True),
)
def _sc_kernel(x_hbm, out_hbm, buf0, buf1, acc, sem0, sem1):
    bufs = (buf0, buf1)
    sems = (sem0, sem1)
    wid = lax.axis_index("s") * 2 + lax.axis_index("c")

    def start_chunk(g, c, b):
        pltpu.async_copy(
            x_hbm.at[pl.ds(g * GROUP_ROWS, GROUP_ROWS), pl.ds(c * CHW, CHW)],
            bufs[b], sems[b])

    def wait_chunk(b):
        pltpu.make_async_copy(
            x_hbm.at[pl.ds(0, GROUP_ROWS), pl.ds(0, CHW)],
            bufs[b], sems[b]).wait()

    def accum_chunk(b):
        buf = bufs[b]
        for s in range(GROUP_ROWS):
            def h_body(h, carry):
                v = buf[s, pl.ds(h * 16, 16)]
                for p in range(1, CQ):
                    v = v + buf[s, pl.ds(p * OUT + h * 16, 16)]
                plsc.addupdate(acc.at[s, pl.ds(h * 16, 16)], v)
                return carry

            lax.fori_loop(0, OUT // 16, h_body, 0, unroll=8)

    zero16 = jnp.zeros((16,), jnp.float32)

    def group_body(gi, carry):
        g = wid * GPW + gi
        start_chunk(g, 0, 0)
        start_chunk(g, 1, 1)
        for s in range(GROUP_ROWS):
            def z_body(h, carry0):
                acc[s, pl.ds(h * 16, 16)] = zero16
                return carry0

            lax.fori_loop(0, OUT // 16, z_body, 0, unroll=8)

        def chunk_body(i, carry2):
            for b in range(2):
                c = 2 * i + b
                wait_chunk(b)
                accum_chunk(b)

                @pl.when(c + 2 < NCH)
                def _():
                    start_chunk(g, c + 2, b)

            return carry2

        lax.fori_loop(0, NCH // 2, chunk_body, 0, unroll=False)
        pltpu.sync_copy(acc, out_hbm.at[pl.ds(g * GROUP_ROWS, GROUP_ROWS), :])
        return carry

    lax.fori_loop(0, GPW, group_body, 0, unroll=False)


_TAIL_BBLK = 256


def _tail_body(x_ref, o_ref):
    x = x_ref[...]
    col = jax.lax.broadcasted_iota(jnp.int32, x.shape, 1)
    o_ref[...] = x[:, 0:OUT] + jnp.where(col < TAIL_COLS, x, 0.0)[:, OUT:2 * OUT]


def _tail_partial(x):
    return pl.pallas_call(
        _tail_body,
        grid=(SC_ROWS // _TAIL_BBLK,),
        in_specs=[pl.BlockSpec((_TAIL_BBLK, 2 * OUT),
                               lambda i: (i, SC_COLS // (2 * OUT)))],
        out_specs=pl.BlockSpec((_TAIL_BBLK, OUT), lambda i: (i, 0)),
        out_shape=jax.ShapeDtypeStruct((SC_ROWS, OUT), jnp.float32),
        compiler_params=pltpu.CompilerParams(
            dimension_semantics=("parallel",),
        ),
    )(x)


TC_BBLK = 256
TC_PER_STEP = 8
TC_CHUNK = TC_PER_STEP * OUT
TC_NK = (IN + TC_CHUNK - 1) // TC_CHUNK
TC_ROW0 = SC_ROWS // TC_BBLK


def _tc_reduce(x):
    acc = x[:, 0:OUT]
    for p in range(1, TC_PER_STEP):
        acc = acc + x[:, p * OUT:(p + 1) * OUT]
    return acc


def _tc_body(x_ref, o_ref):
    k = pl.program_id(1)

    @pl.when(k == 0)
    def _init():
        o_ref[...] = _tc_reduce(x_ref[...])

    @pl.when(jnp.logical_and(k > 0, k < TC_NK - 1))
    def _accum():
        o_ref[...] += _tc_reduce(x_ref[...])

    @pl.when(k == TC_NK - 1)
    def _tail():
        x = x_ref[...]
        col = k * TC_CHUNK + jax.lax.broadcasted_iota(jnp.int32, x.shape, 1)
        o_ref[...] += _tc_reduce(jnp.where(col < IN, x, 0.0))


def _tc_main(x):
    return pl.pallas_call(
        _tc_body,
        grid=((BATCH - SC_ROWS) // TC_BBLK, TC_NK),
        in_specs=[pl.BlockSpec((TC_BBLK, TC_CHUNK),
                               lambda i, k: (i + TC_ROW0, k))],
        out_specs=pl.BlockSpec((TC_BBLK, OUT), lambda i, k: (i, 0)),
        out_shape=jax.ShapeDtypeStruct((BATCH - SC_ROWS, OUT), jnp.float32),
        compiler_params=pltpu.CompilerParams(
            dimension_semantics=("parallel", "arbitrary"),
        ),
    )(x)


@jax.jit
def kernel(probability_distribution):
    x = probability_distribution
    sc_out = _sc_kernel(x)
    tc_out = _tc_main(x)
    tail = _tail_partial(x)
    return jnp.concatenate([sc_out + tail, tc_out], axis=0)


# speedup vs baseline: 1.1585x; 1.0505x over previous
import jax
import jax.numpy as jnp
from jax.experimental import pallas as pl
from jax.experimental.pallas import tpu as pltpu

BATCH = 1024
IN = 100000
OUT = 1024

NOPS = 4
SUBW = 2 * OUT
CHUNK = NOPS * SUBW
NK = (IN + CHUNK - 1) // CHUNK
BBLK = 256
NB = BATCH // BBLK


def _body(x0, x1, x2, x3, o_ref):
    k = pl.program_id(1)
    xs = (x0, x1, x2, x3)

    def psum(j):
        x = xs[j][...]
        col = (k * CHUNK + j * SUBW
               + jax.lax.broadcasted_iota(jnp.int32, (BBLK, SUBW), 1))
        x = jnp.where(col < IN, x, 0.0)
        return x[:, 0:OUT] + x[:, OUT:2 * OUT]

    def psum_nomask(j):
        x = xs[j][...]
        return x[:, 0:OUT] + x[:, OUT:2 * OUT]

    @pl.when(k == 0)
    def _init():
        acc = psum_nomask(0)
        for j in range(1, NOPS):
            acc = acc + psum_nomask(j)
        o_ref[...] = acc

    @pl.when(jnp.logical_and(k > 0, k < NK - 1))
    def _accum():
        acc = psum_nomask(0)
        for j in range(1, NOPS):
            acc = acc + psum_nomask(j)
        o_ref[...] += acc

    @pl.when(k == NK - 1)
    def _tail():
        acc = psum(0)
        for j in range(1, NOPS):
            acc = acc + psum(j)
        o_ref[...] += acc


_MAX_BLK = (IN - 1) // SUBW


def _make_spec(j):
    return pl.BlockSpec(
        (BBLK, SUBW),
        lambda i, k, j=j: (i, jnp.minimum(k * NOPS + j, _MAX_BLK)))


@jax.jit
def kernel(probability_distribution):
    x = probability_distribution
    return pl.pallas_call(
        _body,
        grid=(NB, NK),
        in_specs=[_make_spec(j) for j in range(NOPS)],
        out_specs=pl.BlockSpec((BBLK, OUT), lambda i, k: (i, 0)),
        out_shape=jax.ShapeDtypeStruct((BATCH, OUT), jnp.float32),
        compiler_params=pltpu.CompilerParams(
            dimension_semantics=("parallel", "arbitrary"),
        ),
    )(x, x, x, x)
